# baseline (speedup 1.0000x reference)
import functools

import jax
import jax.numpy as jnp
from jax import lax
from jax.experimental import pallas as pl
from jax.experimental.pallas import tpu as pltpu
from jax.experimental.pallas import tpu_sc as plsc

N = 10000
E = 320000
DIM = 128
G = 128
BN_EPS = 1e-5

NC = 2
NS = 16
NW = NC * NS
CHUNK = 80
NCHUNK = 128
NSTAGE = 1
SCHUNK = NCHUNK // NSTAGE
EPAD = NW * NCHUNK * CHUNK
NPAD = 10240
ROWS_PER_S = NPAD // NS
NBUF = 1


def _sc_segment_sum(x, e4, zrows):
    mesh = plsc.VectorSubcoreMesh(core_axis_name="c", subcore_axis_name="s")

    @functools.partial(
        pl.kernel,
        mesh=mesh,
        out_type=jax.ShapeDtypeStruct((NC, NPAD, DIM), jnp.float32),
        scratch_types=[
            pltpu.VMEM((SCHUNK, CHUNK), jnp.int32),
            pltpu.VMEM((SCHUNK, CHUNK), jnp.int32),
            *[pltpu.VMEM((CHUNK, DIM), jnp.float32) for _ in range(NBUF)],
            pltpu.VMEM_SHARED((NPAD, DIM), jnp.float32),
            *[pltpu.SemaphoreType.DMA for _ in range(NBUF)],
        ],
    )
    def k(x_hbm, e_hbm, z_hbm, out_hbm, src_v, dst_v, *rest):
        rows = rest[:NBUF]
        acc = rest[NBUF]
        sems = rest[NBUF + 1:]
        c = lax.axis_index("c")
        s = lax.axis_index("s")
        wid = s * NC + c

        pltpu.sync_copy(z_hbm, acc.at[pl.ds(s * ROWS_PER_S, ROWS_PER_S)])

        plsc.subcore_barrier()

        for st in range(NSTAGE):
            if NSTAGE == 1:
                pltpu.sync_copy(e_hbm.at[0, wid], src_v)
                pltpu.sync_copy(e_hbm.at[1, wid], dst_v)
            else:
                pltpu.sync_copy(e_hbm.at[0, wid, pl.ds(st * SCHUNK, SCHUNK)], src_v)
                pltpu.sync_copy(e_hbm.at[1, wid, pl.ds(st * SCHUNK, SCHUNK)], dst_v)

            @pl.loop(0, SCHUNK)
            def _(j):
                pltpu.async_copy(x_hbm.at[src_v.at[j]], rows[0], sems[0]).wait()
                pltpu.sync_copy(rows[0], acc.at[dst_v.at[j]], add=True)

        plsc.subcore_barrier()

        pltpu.sync_copy(
            acc.at[pl.ds(s * ROWS_PER_S, ROWS_PER_S)],
            out_hbm.at[c, pl.ds(s * ROWS_PER_S, ROWS_PER_S)],
        )

    return k(x, e4, zrows)


def _tc_layer(x, agg, w1, b1, w2, b2, gamma, beta):
    def body(x_ref, a_ref, w1_ref, b1_ref, w2_ref, b2_ref, g_ref, bt_ref, o_ref):
        h = x_ref[...] + a_ref[0, :N] + a_ref[1, :N]
        h = jnp.dot(h, w1_ref[...], preferred_element_type=jnp.float32) + b1_ref[...]
        h = jnp.maximum(h, 0.0)
        h = jnp.dot(h, w2_ref[...], preferred_element_type=jnp.float32) + b2_ref[...]
        a = jnp.maximum(h, 0.0)
        mean = jnp.mean(a, axis=0, keepdims=True)
        var = jnp.mean((a - mean) ** 2, axis=0, keepdims=True)
        o_ref[...] = (a - mean) * lax.rsqrt(var + BN_EPS) * g_ref[...] + bt_ref[...]

    return pl.pallas_call(
        body,
        out_shape=jax.ShapeDtypeStruct((N, DIM), jnp.float32),
    )(x, agg, w1, b1, w2, b2, gamma, beta)


def _bn_in_kernel(a, g, bt):
    mean = jnp.mean(a, axis=0, keepdims=True)
    var = jnp.mean((a - mean) ** 2, axis=0, keepdims=True)
    return (a - mean) * lax.rsqrt(var + BN_EPS) * g + bt


def _tc_final(x, batch2d, sw, sb, heads):
    def body(x_ref, b_ref, sw_ref, sb_ref,
             nmw, nmb, nmg, nmbt, nlw, nlb, nlg, nlbt,
             gmw, gmb, gmg, gmbt, glw, glb, glg, glbt,
             o_nm, o_nl, o_gm, o_gl):
        x_ = x_ref[...]
        logit = jnp.sum(x_ * sw_ref[...], axis=1, keepdims=True) + sb_ref[...]
        w = jax.nn.sigmoid(logit)
        summary = w * x_
        noisy = x_ - summary

        ids = b_ref[...]
        row = lax.broadcasted_iota(jnp.int32, (G, N), 0)
        onehot = (row == ids).astype(jnp.float32)
        slots = jnp.dot(onehot, summary, preferred_element_type=jnp.float32)

        nm = jnp.maximum(jnp.dot(noisy, nmw[...], preferred_element_type=jnp.float32) + nmb[...], 0.0)
        o_nm[...] = _bn_in_kernel(nm, nmg[...], nmbt[...])
        nl = jnp.maximum(jnp.dot(noisy, nlw[...], preferred_element_type=jnp.float32) + nlb[...], 0.0)
        o_nl[...] = _bn_in_kernel(nl, nlg[...], nlbt[...])
        gm = jnp.maximum(jnp.dot(slots, gmw[...], preferred_element_type=jnp.float32) + gmb[...], 0.0)
        o_gm[...] = _bn_in_kernel(gm, gmg[...], gmbt[...])
        gl = jnp.maximum(jnp.dot(slots, glw[...], preferred_element_type=jnp.float32) + glb[...], 0.0)
        o_gl[...] = _bn_in_kernel(gl, glg[...], glbt[...])

    flat_heads = [t for h in heads for t in h]
    return pl.pallas_call(
        body,
        out_shape=(
            jax.ShapeDtypeStruct((N, DIM), jnp.float32),
            jax.ShapeDtypeStruct((N, DIM), jnp.float32),
            jax.ShapeDtypeStruct((G, DIM), jnp.float32),
            jax.ShapeDtypeStruct((G, DIM), jnp.float32),
        ),
    )(x, batch2d, sw, sb, *flat_heads)


def _row(v):
    return v.reshape(1, -1)


def kernel(x, edge_index, batch, params):
    npad_e = EPAD - E
    pad = jnp.stack([jnp.zeros((npad_e,), jnp.int32),
                     N + jnp.arange(npad_e, dtype=jnp.int32) % (NPAD - N)])
    e4 = jnp.concatenate([edge_index, pad], axis=1).reshape(2, NW, NCHUNK, CHUNK)
    zrows = jnp.zeros((ROWS_PER_S, DIM), jnp.float32)

    for i in range(3):
        c = params["convs"][i]
        bn = params["bns"][i]
        agg = _sc_segment_sum(x, e4, zrows)
        x = _tc_layer(x, agg, c["W1"], _row(c["b1"]), c["W2"], _row(c["b2"]),
                      _row(bn["gamma"]), _row(bn["beta"]))

    heads = []
    for name in ["node_mu", "node_logvar", "graph_mu", "graph_logvar"]:
        heads.append((params[name + "_W"], _row(params[name + "_b"]),
                      _row(params[name + "_gamma"]), _row(params[name + "_beta"])))

    return _tc_final(x, _row(batch), _row(params["summary_W"][:, 0]),
                     _row(params["summary_b"]), heads)


# speedup vs baseline: 2.4030x; 2.4030x over previous
import functools

import jax
import jax.numpy as jnp
from jax import lax
from jax.experimental import pallas as pl
from jax.experimental.pallas import tpu as pltpu
from jax.experimental.pallas import tpu_sc as plsc

N = 10000
E = 320000
DIM = 128
G = 128
BN_EPS = 1e-5

NC = 2
NS = 16
NW = NC * NS
CHUNK = 80
NCHUNK = 128
NSTAGE = 1
SCHUNK = NCHUNK // NSTAGE
EPAD = NW * NCHUNK * CHUNK
NPAD = 10240
ROWS_PER_S = NPAD // NS
NBUF = 1


def _sc_segment_sum(x, e4, zrows):
    mesh = plsc.VectorSubcoreMesh(core_axis_name="c", subcore_axis_name="s")

    @functools.partial(
        pl.kernel,
        mesh=mesh,
        out_type=jax.ShapeDtypeStruct((NC, NPAD, DIM), jnp.float32),
        scratch_types=[
            pltpu.VMEM((SCHUNK, CHUNK), jnp.int32),
            pltpu.VMEM((SCHUNK, CHUNK), jnp.int32),
            *[pltpu.VMEM((CHUNK, DIM), jnp.float32) for _ in range(NBUF)],
            pltpu.VMEM_SHARED((NPAD, DIM), jnp.float32),
            *[pltpu.SemaphoreType.DMA for _ in range(NBUF)],
        ],
    )
    def k(x_hbm, e_hbm, z_hbm, out_hbm, src_v, dst_v, *rest):
        rows = rest[:NBUF]
        acc = rest[NBUF]
        sems = rest[NBUF + 1:]
        c = lax.axis_index("c")
        s = lax.axis_index("s")
        wid = s * NC + c

        pltpu.sync_copy(z_hbm, acc.at[pl.ds(s * ROWS_PER_S, ROWS_PER_S)])

        plsc.subcore_barrier()

        for st in range(NSTAGE):
            if NSTAGE == 1:
                pltpu.sync_copy(e_hbm.at[0, wid], src_v)
                pltpu.sync_copy(e_hbm.at[1, wid], dst_v)
            else:
                pltpu.sync_copy(e_hbm.at[0, wid, pl.ds(st * SCHUNK, SCHUNK)], src_v)
                pltpu.sync_copy(e_hbm.at[1, wid, pl.ds(st * SCHUNK, SCHUNK)], dst_v)

            @pl.loop(0, SCHUNK)
            def _(j):
                pltpu.async_copy(x_hbm.at[src_v.at[j]], rows[0], sems[0]).wait()
                pltpu.sync_copy(rows[0], acc.at[dst_v.at[j]], add=True)

        plsc.subcore_barrier()

        pltpu.sync_copy(
            acc.at[pl.ds(s * ROWS_PER_S, ROWS_PER_S)],
            out_hbm.at[c, pl.ds(s * ROWS_PER_S, ROWS_PER_S)],
        )

    return k(x, e4, zrows)


def _tc_layer(x, agg, w1, b1, w2, b2, gamma, beta):
    def body(x_ref, a_ref, w1_ref, b1_ref, w2_ref, b2_ref, g_ref, bt_ref, o_ref):
        h = x_ref[...] + a_ref[0, :N] + a_ref[1, :N]
        h = jnp.dot(h, w1_ref[...], preferred_element_type=jnp.float32) + b1_ref[...]
        h = jnp.maximum(h, 0.0)
        h = jnp.dot(h, w2_ref[...], preferred_element_type=jnp.float32) + b2_ref[...]
        a = jnp.maximum(h, 0.0)
        mean = jnp.mean(a, axis=0, keepdims=True)
        var = jnp.mean((a - mean) ** 2, axis=0, keepdims=True)
        o_ref[...] = (a - mean) * lax.rsqrt(var + BN_EPS) * g_ref[...] + bt_ref[...]

    return pl.pallas_call(
        body,
        out_shape=jax.ShapeDtypeStruct((N, DIM), jnp.float32),
    )(x, agg, w1, b1, w2, b2, gamma, beta)


def _bn_in_kernel(a, g, bt):
    mean = jnp.mean(a, axis=0, keepdims=True)
    var = jnp.mean((a - mean) ** 2, axis=0, keepdims=True)
    return (a - mean) * lax.rsqrt(var + BN_EPS) * g + bt


def _tc_final(x, batch2d, sw, sb, heads):
    def body(x_ref, b_ref, sw_ref, sb_ref,
             nmw, nmb, nmg, nmbt, nlw, nlb, nlg, nlbt,
             gmw, gmb, gmg, gmbt, glw, glb, glg, glbt,
             o_nm, o_nl, o_gm, o_gl):
        x_ = x_ref[...]
        logit = jnp.sum(x_ * sw_ref[...], axis=1, keepdims=True) + sb_ref[...]
        w = jax.nn.sigmoid(logit)
        summary = w * x_
        noisy = x_ - summary

        ids = b_ref[...]
        row = lax.broadcasted_iota(jnp.int32, (G, N), 0)
        onehot = (row == ids).astype(jnp.float32)
        slots = jnp.dot(onehot, summary, preferred_element_type=jnp.float32)

        nm = jnp.maximum(jnp.dot(noisy, nmw[...], preferred_element_type=jnp.float32) + nmb[...], 0.0)
        o_nm[...] = _bn_in_kernel(nm, nmg[...], nmbt[...])
        nl = jnp.maximum(jnp.dot(noisy, nlw[...], preferred_element_type=jnp.float32) + nlb[...], 0.0)
        o_nl[...] = _bn_in_kernel(nl, nlg[...], nlbt[...])
        gm = jnp.maximum(jnp.dot(slots, gmw[...], preferred_element_type=jnp.float32) + gmb[...], 0.0)
        o_gm[...] = _bn_in_kernel(gm, gmg[...], gmbt[...])
        gl = jnp.maximum(jnp.dot(slots, glw[...], preferred_element_type=jnp.float32) + glb[...], 0.0)
        o_gl[...] = _bn_in_kernel(gl, glg[...], glbt[...])

    flat_heads = [t for h in heads for t in h]
    return pl.pallas_call(
        body,
        out_shape=(
            jax.ShapeDtypeStruct((N, DIM), jnp.float32),
            jax.ShapeDtypeStruct((N, DIM), jnp.float32),
            jax.ShapeDtypeStruct((G, DIM), jnp.float32),
            jax.ShapeDtypeStruct((G, DIM), jnp.float32),
        ),
    )(x, batch2d, sw, sb, *flat_heads)


def _row(v):
    return v.reshape(1, -1)


def kernel(x, edge_index, batch, params):
    npad_e = EPAD - E
    ar = jnp.arange(npad_e, dtype=jnp.int32)
    pad = jnp.stack([ar % N, N + ar % (NPAD - N)])
    e4 = jnp.concatenate([edge_index, pad], axis=1).reshape(2, NW, NCHUNK, CHUNK)
    zrows = jnp.zeros((ROWS_PER_S, DIM), jnp.float32)

    for i in range(3):
        c = params["convs"][i]
        bn = params["bns"][i]
        agg = _sc_segment_sum(x, e4, zrows)
        x = _tc_layer(x, agg, c["W1"], _row(c["b1"]), c["W2"], _row(c["b2"]),
                      _row(bn["gamma"]), _row(bn["beta"]))

    heads = []
    for name in ["node_mu", "node_logvar", "graph_mu", "graph_logvar"]:
        heads.append((params[name + "_W"], _row(params[name + "_b"]),
                      _row(params[name + "_gamma"]), _row(params[name + "_beta"])))

    return _tc_final(x, _row(batch), _row(params["summary_W"][:, 0]),
                     _row(params["summary_b"]), heads)


# speedup vs baseline: 3.8080x; 1.5847x over previous
import functools

import jax
import jax.numpy as jnp
from jax import lax
from jax.experimental import pallas as pl
from jax.experimental.pallas import tpu as pltpu
from jax.experimental.pallas import tpu_sc as plsc

N = 10000
E = 320000
DIM = 128
G = 128
BN_EPS = 1e-5

NC = 2
NS = 16
NW = NC * NS
CHUNK = 80
NCHUNK = 128
NSTAGE = 2
SCHUNK = NCHUNK // NSTAGE
EPAD = NW * NCHUNK * CHUNK
NPAD = 10240
ROWS_PER_S = NPAD // NS
NBUF = 2


def _sc_segment_sum(x, e4, zrows):
    mesh = plsc.VectorSubcoreMesh(core_axis_name="c", subcore_axis_name="s")

    @functools.partial(
        pl.kernel,
        mesh=mesh,
        out_type=jax.ShapeDtypeStruct((NC, NPAD, DIM), jnp.float32),
        scratch_types=[
            pltpu.VMEM((SCHUNK, CHUNK), jnp.int32),
            pltpu.VMEM((SCHUNK, CHUNK), jnp.int32),
            *[pltpu.VMEM((CHUNK, DIM), jnp.float32) for _ in range(NBUF)],
            pltpu.VMEM_SHARED((NPAD, DIM), jnp.float32),
            *[pltpu.SemaphoreType.DMA for _ in range(NBUF)],
        ],
    )
    def k(x_hbm, e_hbm, z_hbm, out_hbm, src_v, dst_v, *rest):
        rows = rest[:NBUF]
        acc = rest[NBUF]
        sems = rest[NBUF + 1:]
        c = lax.axis_index("c")
        s = lax.axis_index("s")
        wid = s * NC + c

        pltpu.sync_copy(z_hbm, acc.at[pl.ds(s * ROWS_PER_S, ROWS_PER_S)])

        plsc.subcore_barrier()

        for st in range(NSTAGE):
            if NSTAGE == 1:
                pltpu.sync_copy(e_hbm.at[0, wid], src_v)
                pltpu.sync_copy(e_hbm.at[1, wid], dst_v)
            else:
                pltpu.sync_copy(e_hbm.at[0, wid, pl.ds(st * SCHUNK, SCHUNK)], src_v)
                pltpu.sync_copy(e_hbm.at[1, wid, pl.ds(st * SCHUNK, SCHUNK)], dst_v)

            for b in range(NBUF):
                pltpu.async_copy(x_hbm.at[src_v.at[b]], rows[b], sems[b])

            @pl.loop(0, SCHUNK // NBUF)
            def _(jj):
                j0 = jj * NBUF
                for b in range(NBUF):
                    j = j0 + b
                    pltpu.make_async_copy(x_hbm.at[src_v.at[j]], rows[b], sems[b]).wait()
                    pltpu.sync_copy(rows[b], acc.at[dst_v.at[j]], add=True)

                    @pl.when(j + NBUF < SCHUNK)
                    def _():
                        pltpu.async_copy(x_hbm.at[src_v.at[j + NBUF]], rows[b], sems[b])

        plsc.subcore_barrier()

        pltpu.sync_copy(
            acc.at[pl.ds(s * ROWS_PER_S, ROWS_PER_S)],
            out_hbm.at[c, pl.ds(s * ROWS_PER_S, ROWS_PER_S)],
        )

    return k(x, e4, zrows)


def _tc_layer(x, agg, w1, b1, w2, b2, gamma, beta):
    def body(x_ref, a_ref, w1_ref, b1_ref, w2_ref, b2_ref, g_ref, bt_ref, o_ref):
        h = x_ref[...] + a_ref[0, :N] + a_ref[1, :N]
        h = jnp.dot(h, w1_ref[...], preferred_element_type=jnp.float32) + b1_ref[...]
        h = jnp.maximum(h, 0.0)
        h = jnp.dot(h, w2_ref[...], preferred_element_type=jnp.float32) + b2_ref[...]
        a = jnp.maximum(h, 0.0)
        mean = jnp.mean(a, axis=0, keepdims=True)
        var = jnp.mean((a - mean) ** 2, axis=0, keepdims=True)
        o_ref[...] = (a - mean) * lax.rsqrt(var + BN_EPS) * g_ref[...] + bt_ref[...]

    return pl.pallas_call(
        body,
        out_shape=jax.ShapeDtypeStruct((N, DIM), jnp.float32),
    )(x, agg, w1, b1, w2, b2, gamma, beta)


def _bn_in_kernel(a, g, bt):
    mean = jnp.mean(a, axis=0, keepdims=True)
    var = jnp.mean((a - mean) ** 2, axis=0, keepdims=True)
    return (a - mean) * lax.rsqrt(var + BN_EPS) * g + bt


def _tc_final(x, batch2d, sw, sb, heads):
    def body(x_ref, b_ref, sw_ref, sb_ref,
             nmw, nmb, nmg, nmbt, nlw, nlb, nlg, nlbt,
             gmw, gmb, gmg, gmbt, glw, glb, glg, glbt,
             o_nm, o_nl, o_gm, o_gl):
        x_ = x_ref[...]
        logit = jnp.sum(x_ * sw_ref[...], axis=1, keepdims=True) + sb_ref[...]
        w = jax.nn.sigmoid(logit)
        summary = w * x_
        noisy = x_ - summary

        ids = b_ref[...]
        row = lax.broadcasted_iota(jnp.int32, (G, N), 0)
        onehot = (row == ids).astype(jnp.float32)
        slots = jnp.dot(onehot, summary, preferred_element_type=jnp.float32)

        nm = jnp.maximum(jnp.dot(noisy, nmw[...], preferred_element_type=jnp.float32) + nmb[...], 0.0)
        o_nm[...] = _bn_in_kernel(nm, nmg[...], nmbt[...])
        nl = jnp.maximum(jnp.dot(noisy, nlw[...], preferred_element_type=jnp.float32) + nlb[...], 0.0)
        o_nl[...] = _bn_in_kernel(nl, nlg[...], nlbt[...])
        gm = jnp.maximum(jnp.dot(slots, gmw[...], preferred_element_type=jnp.float32) + gmb[...], 0.0)
        o_gm[...] = _bn_in_kernel(gm, gmg[...], gmbt[...])
        gl = jnp.maximum(jnp.dot(slots, glw[...], preferred_element_type=jnp.float32) + glb[...], 0.0)
        o_gl[...] = _bn_in_kernel(gl, glg[...], glbt[...])

    flat_heads = [t for h in heads for t in h]
    return pl.pallas_call(
        body,
        out_shape=(
            jax.ShapeDtypeStruct((N, DIM), jnp.float32),
            jax.ShapeDtypeStruct((N, DIM), jnp.float32),
            jax.ShapeDtypeStruct((G, DIM), jnp.float32),
            jax.ShapeDtypeStruct((G, DIM), jnp.float32),
        ),
    )(x, batch2d, sw, sb, *flat_heads)


def _row(v):
    return v.reshape(1, -1)


def kernel(x, edge_index, batch, params):
    npad_e = EPAD - E
    ar = jnp.arange(npad_e, dtype=jnp.int32)
    pad = jnp.stack([ar % N, N + ar % (NPAD - N)])
    e4 = jnp.concatenate([edge_index, pad], axis=1).reshape(2, NW, NCHUNK, CHUNK)
    zrows = jnp.zeros((ROWS_PER_S, DIM), jnp.float32)

    for i in range(3):
        c = params["convs"][i]
        bn = params["bns"][i]
        agg = _sc_segment_sum(x, e4, zrows)
        x = _tc_layer(x, agg, c["W1"], _row(c["b1"]), c["W2"], _row(c["b2"]),
                      _row(bn["gamma"]), _row(bn["beta"]))

    heads = []
    for name in ["node_mu", "node_logvar", "graph_mu", "graph_logvar"]:
        heads.append((params[name + "_W"], _row(params[name + "_b"]),
                      _row(params[name + "_gamma"]), _row(params[name + "_beta"])))

    return _tc_final(x, _row(batch), _row(params["summary_W"][:, 0]),
                     _row(params["summary_b"]), heads)


# speedup vs baseline: 4.4541x; 1.1697x over previous
import functools

import jax
import jax.numpy as jnp
from jax import lax
from jax.experimental import pallas as pl
from jax.experimental.pallas import tpu as pltpu
from jax.experimental.pallas import tpu_sc as plsc

N = 10000
E = 320000
DIM = 128
G = 128
BN_EPS = 1e-5

NC = 2
NS = 16
NW = NC * NS
CHUNK = 80
NCHUNK = 128
NSTAGE = 2
SCHUNK = NCHUNK // NSTAGE
EPAD = NW * NCHUNK * CHUNK
NPAD = 10240
ROWS_PER_S = NPAD // NS
NBUF = 3


def _sc_segment_sum(x, e4, zrows):
    mesh = plsc.VectorSubcoreMesh(core_axis_name="c", subcore_axis_name="s")

    @functools.partial(
        pl.kernel,
        mesh=mesh,
        out_type=jax.ShapeDtypeStruct((NC, NPAD, DIM), jnp.float32),
        scratch_types=[
            pltpu.VMEM((SCHUNK, CHUNK), jnp.int32),
            pltpu.VMEM((SCHUNK, CHUNK), jnp.int32),
            *[pltpu.VMEM((CHUNK, DIM), jnp.float32) for _ in range(NBUF)],
            pltpu.VMEM_SHARED((NPAD, DIM), jnp.float32),
            *[pltpu.SemaphoreType.DMA for _ in range(NBUF)],
        ],
    )
    def k(x_hbm, e_hbm, z_hbm, out_hbm, src_v, dst_v, *rest):
        rows = rest[:NBUF]
        acc = rest[NBUF]
        sems = rest[NBUF + 1:]
        c = lax.axis_index("c")
        s = lax.axis_index("s")
        wid = s * NC + c

        pltpu.sync_copy(z_hbm, acc.at[pl.ds(s * ROWS_PER_S, ROWS_PER_S)])

        plsc.subcore_barrier()

        for st in range(NSTAGE):
            if NSTAGE == 1:
                pltpu.sync_copy(e_hbm.at[0, wid], src_v)
                pltpu.sync_copy(e_hbm.at[1, wid], dst_v)
            else:
                pltpu.sync_copy(e_hbm.at[0, wid, pl.ds(st * SCHUNK, SCHUNK)], src_v)
                pltpu.sync_copy(e_hbm.at[1, wid, pl.ds(st * SCHUNK, SCHUNK)], dst_v)

            for b in range(NBUF):
                pltpu.async_copy(x_hbm.at[src_v.at[b]], rows[b], sems[b])

            FULL = SCHUNK // NBUF

            @pl.loop(0, FULL)
            def _(jj):
                j0 = jj * NBUF
                for b in range(NBUF):
                    j = j0 + b
                    pltpu.make_async_copy(x_hbm.at[src_v.at[j]], rows[b], sems[b]).wait()
                    pltpu.sync_copy(rows[b], acc.at[dst_v.at[j]], add=True)

                    @pl.when(j + NBUF < SCHUNK)
                    def _():
                        pltpu.async_copy(x_hbm.at[src_v.at[j + NBUF]], rows[b], sems[b])

            for b in range(SCHUNK - FULL * NBUF):
                j = FULL * NBUF + b
                pltpu.make_async_copy(x_hbm.at[src_v.at[j]], rows[b], sems[b]).wait()
                pltpu.sync_copy(rows[b], acc.at[dst_v.at[j]], add=True)

        plsc.subcore_barrier()

        pltpu.sync_copy(
            acc.at[pl.ds(s * ROWS_PER_S, ROWS_PER_S)],
            out_hbm.at[c, pl.ds(s * ROWS_PER_S, ROWS_PER_S)],
        )

    return k(x, e4, zrows)


def _tc_layer(x, agg, w1, b1, w2, b2, gamma, beta):
    def body(x_ref, a_ref, w1_ref, b1_ref, w2_ref, b2_ref, g_ref, bt_ref, o_ref):
        h = x_ref[...] + a_ref[0, :N] + a_ref[1, :N]
        h = jnp.dot(h, w1_ref[...], preferred_element_type=jnp.float32) + b1_ref[...]
        h = jnp.maximum(h, 0.0)
        h = jnp.dot(h, w2_ref[...], preferred_element_type=jnp.float32) + b2_ref[...]
        a = jnp.maximum(h, 0.0)
        mean = jnp.mean(a, axis=0, keepdims=True)
        var = jnp.mean((a - mean) ** 2, axis=0, keepdims=True)
        o_ref[...] = (a - mean) * lax.rsqrt(var + BN_EPS) * g_ref[...] + bt_ref[...]

    return pl.pallas_call(
        body,
        out_shape=jax.ShapeDtypeStruct((N, DIM), jnp.float32),
    )(x, agg, w1, b1, w2, b2, gamma, beta)


def _bn_in_kernel(a, g, bt):
    mean = jnp.mean(a, axis=0, keepdims=True)
    var = jnp.mean((a - mean) ** 2, axis=0, keepdims=True)
    return (a - mean) * lax.rsqrt(var + BN_EPS) * g + bt


def _tc_final(x, batch2d, sw, sb, heads):
    def body(x_ref, b_ref, sw_ref, sb_ref,
             nmw, nmb, nmg, nmbt, nlw, nlb, nlg, nlbt,
             gmw, gmb, gmg, gmbt, glw, glb, glg, glbt,
             o_nm, o_nl, o_gm, o_gl):
        x_ = x_ref[...]
        logit = jnp.sum(x_ * sw_ref[...], axis=1, keepdims=True) + sb_ref[...]
        w = jax.nn.sigmoid(logit)
        summary = w * x_
        noisy = x_ - summary

        ids = b_ref[...]
        row = lax.broadcasted_iota(jnp.int32, (G, N), 0)
        onehot = (row == ids).astype(jnp.float32)
        slots = jnp.dot(onehot, summary, preferred_element_type=jnp.float32)

        nm = jnp.maximum(jnp.dot(noisy, nmw[...], preferred_element_type=jnp.float32) + nmb[...], 0.0)
        o_nm[...] = _bn_in_kernel(nm, nmg[...], nmbt[...])
        nl = jnp.maximum(jnp.dot(noisy, nlw[...], preferred_element_type=jnp.float32) + nlb[...], 0.0)
        o_nl[...] = _bn_in_kernel(nl, nlg[...], nlbt[...])
        gm = jnp.maximum(jnp.dot(slots, gmw[...], preferred_element_type=jnp.float32) + gmb[...], 0.0)
        o_gm[...] = _bn_in_kernel(gm, gmg[...], gmbt[...])
        gl = jnp.maximum(jnp.dot(slots, glw[...], preferred_element_type=jnp.float32) + glb[...], 0.0)
        o_gl[...] = _bn_in_kernel(gl, glg[...], glbt[...])

    flat_heads = [t for h in heads for t in h]
    return pl.pallas_call(
        body,
        out_shape=(
            jax.ShapeDtypeStruct((N, DIM), jnp.float32),
            jax.ShapeDtypeStruct((N, DIM), jnp.float32),
            jax.ShapeDtypeStruct((G, DIM), jnp.float32),
            jax.ShapeDtypeStruct((G, DIM), jnp.float32),
        ),
    )(x, batch2d, sw, sb, *flat_heads)


def _row(v):
    return v.reshape(1, -1)


def kernel(x, edge_index, batch, params):
    npad_e = EPAD - E
    ar = jnp.arange(npad_e, dtype=jnp.int32)
    pad = jnp.stack([ar % N, N + ar % (NPAD - N)])
    e4 = jnp.concatenate([edge_index, pad], axis=1).reshape(2, NW, NCHUNK, CHUNK)
    zrows = jnp.zeros((ROWS_PER_S, DIM), jnp.float32)

    for i in range(3):
        c = params["convs"][i]
        bn = params["bns"][i]
        agg = _sc_segment_sum(x, e4, zrows)
        x = _tc_layer(x, agg, c["W1"], _row(c["b1"]), c["W2"], _row(c["b2"]),
                      _row(bn["gamma"]), _row(bn["beta"]))

    heads = []
    for name in ["node_mu", "node_logvar", "graph_mu", "graph_logvar"]:
        heads.append((params[name + "_W"], _row(params[name + "_b"]),
                      _row(params[name + "_gamma"]), _row(params[name + "_beta"])))

    return _tc_final(x, _row(batch), _row(params["summary_W"][:, 0]),
                     _row(params["summary_b"]), heads)
